# baseline (speedup 1.0000x reference)
import functools

import jax
import jax.numpy as jnp
from jax import lax
from jax.experimental import pallas as pl
from jax.experimental.pallas import tpu as pltpu
from jax.experimental.pallas import tpu_sc as plsc

N = 10000
E = 320000
DF = 128
H = 64
NCORES = 2
NSUB = 16
NW = NCORES * NSUB
EPT = E // NW
CH = 80
KCH = EPT // CH
NPAD = 10240
RPT = NPAD // NSUB
RSLAB = 128
NSLAB = RPT // RSLAB


def _sc_mesh():
    return plsc.VectorSubcoreMesh(
        core_axis_name="c", subcore_axis_name="s",
        num_cores=NCORES, num_subcores=NSUB)


_SC_PARAMS = pltpu.CompilerParams(use_tc_tiling_on_sc=False)


def _fill_rows(ref, nrows, ncols, value):
    vec = jnp.full((16,), value, jnp.float32)

    def body(i, carry):
        for k in range(ncols // 16):
            ref[i, pl.ds(k * 16, 16)] = vec
        return carry

    lax.fori_loop(0, nrows, body, 0)


def _make_edge_pass0():
    out_type = [
        jax.ShapeDtypeStruct((NCORES, NPAD, H), jnp.float32),
        jax.ShapeDtypeStruct((NCORES, NPAD, 16), jnp.float32),
        jax.ShapeDtypeStruct((NCORES, NPAD, 16), jnp.float32),
    ]
    scratch = [
        pltpu.VMEM((EPT,), jnp.int32),
        pltpu.VMEM((EPT,), jnp.int32),
        pltpu.VMEM((2, CH, H), jnp.float32),
        pltpu.VMEM((2, CH, 16), jnp.float32),
        pltpu.VMEM((CH, 16), jnp.float32),
        pltpu.VMEM((RSLAB, H), jnp.float32),
        pltpu.VMEM((RSLAB, 16), jnp.float32),
        pltpu.VMEM_SHARED((NPAD, H), jnp.float32),
        pltpu.VMEM_SHARED((NPAD, 16), jnp.float32),
        pltpu.VMEM_SHARED((NPAD, 16), jnp.float32),
        pltpu.SemaphoreType.DMA((2,)),
        pltpu.SemaphoreType.DMA((2,)),
    ]

    @functools.partial(pl.kernel, out_type=out_type, mesh=_sc_mesh(),
                       scratch_types=scratch, compiler_params=_SC_PARAMS)
    def edge_pass0(ei_hbm, ef_hbm, g_hbm,
                   s_out, e_out, d_out,
                   src_v, dst_v, rows_v, ef_v, ones_v, st_w, st_n,
                   s_sh, e_sh, d_sh, gsem, esem):
        c = lax.axis_index("c")
        s = lax.axis_index("s")
        wid = s * NCORES + c
        ebase = pl.multiple_of(wid * EPT, CH)
        base = pl.multiple_of(s * RPT, RSLAB)
        _fill_rows(st_w, RSLAB, H, 0.0)
        _fill_rows(st_n, RSLAB, 16, 0.0)
        for k in range(NSLAB):
            r0 = base + k * RSLAB
            pltpu.sync_copy(st_w, s_sh.at[pl.ds(r0, RSLAB), :])
            pltpu.sync_copy(st_n, e_sh.at[pl.ds(r0, RSLAB), :])
            pltpu.sync_copy(st_n, d_sh.at[pl.ds(r0, RSLAB), :])
        _fill_rows(ones_v, CH, 16, 1.0)
        plsc.subcore_barrier()

        pltpu.sync_copy(ei_hbm.at[0, pl.ds(ebase, EPT)], src_v)
        pltpu.sync_copy(ei_hbm.at[1, pl.ds(ebase, EPT)], dst_v)

        def start(j, b):
            o = pl.multiple_of(j * CH, CH)
            pltpu.async_copy(g_hbm.at[src_v.at[pl.ds(o, CH)]], rows_v.at[b],
                             gsem.at[b])
            pltpu.async_copy(ef_hbm.at[pl.ds(ebase + o, CH), :], ef_v.at[b],
                             esem.at[b])

        def finish(j, b):
            o = pl.multiple_of(j * CH, CH)
            pltpu.make_async_copy(g_hbm.at[src_v.at[pl.ds(o, CH)]],
                                  rows_v.at[b], gsem.at[b]).wait()
            pltpu.make_async_copy(ef_hbm.at[pl.ds(ebase + o, CH), :],
                                  ef_v.at[b], esem.at[b]).wait()
            dsts = dst_v.at[pl.ds(o, CH)]
            pltpu.sync_copy(rows_v.at[b], s_sh.at[dsts], add=True)
            pltpu.sync_copy(ef_v.at[b], e_sh.at[dsts], add=True)
            pltpu.sync_copy(ones_v, d_sh.at[dsts], add=True)

        start(0, 0)

        def chunk(j, carry):
            b = lax.rem(j, 2)

            @pl.when(j < KCH - 1)
            def _():
                start(j + 1, 1 - b)

            finish(j, b)
            return carry

        lax.fori_loop(0, KCH, chunk, 0)
        plsc.subcore_barrier()

        for k in range(NSLAB):
            r0 = base + k * RSLAB
            pltpu.sync_copy(s_sh.at[pl.ds(r0, RSLAB), :], st_w)
            pltpu.sync_copy(st_w, s_out.at[c, pl.ds(r0, RSLAB), :])
            pltpu.sync_copy(e_sh.at[pl.ds(r0, RSLAB), :], st_n)
            pltpu.sync_copy(st_n, e_out.at[c, pl.ds(r0, RSLAB), :])
            pltpu.sync_copy(d_sh.at[pl.ds(r0, RSLAB), :], st_n)
            pltpu.sync_copy(st_n, d_out.at[c, pl.ds(r0, RSLAB), :])

    return edge_pass0


def _make_edge_pass(width):
    out_type = jax.ShapeDtypeStruct((NCORES, NPAD, width), jnp.float32)
    scratch = [
        pltpu.VMEM((EPT,), jnp.int32),
        pltpu.VMEM((EPT,), jnp.int32),
        pltpu.VMEM((2, CH, width), jnp.float32),
        pltpu.VMEM((RSLAB, width), jnp.float32),
        pltpu.VMEM_SHARED((NPAD, width), jnp.float32),
        pltpu.SemaphoreType.DMA((2,)),
    ]

    @functools.partial(pl.kernel, out_type=out_type, mesh=_sc_mesh(),
                       scratch_types=scratch, compiler_params=_SC_PARAMS)
    def edge_pass(ei_hbm, g_hbm, s_out,
                  src_v, dst_v, rows_v, st_v, s_sh, gsem):
        c = lax.axis_index("c")
        s = lax.axis_index("s")
        wid = s * NCORES + c
        ebase = pl.multiple_of(wid * EPT, CH)
        base = pl.multiple_of(s * RPT, RSLAB)
        _fill_rows(st_v, RSLAB, width, 0.0)
        for k in range(NSLAB):
            pltpu.sync_copy(st_v, s_sh.at[pl.ds(base + k * RSLAB, RSLAB), :])
        plsc.subcore_barrier()

        pltpu.sync_copy(ei_hbm.at[0, pl.ds(ebase, EPT)], src_v)
        pltpu.sync_copy(ei_hbm.at[1, pl.ds(ebase, EPT)], dst_v)

        def start(j, b):
            o = pl.multiple_of(j * CH, CH)
            pltpu.async_copy(g_hbm.at[src_v.at[pl.ds(o, CH)]], rows_v.at[b],
                             gsem.at[b])

        def finish(j, b):
            o = pl.multiple_of(j * CH, CH)
            pltpu.make_async_copy(g_hbm.at[src_v.at[pl.ds(o, CH)]],
                                  rows_v.at[b], gsem.at[b]).wait()
            pltpu.sync_copy(rows_v.at[b], s_sh.at[dst_v.at[pl.ds(o, CH)]],
                            add=True)

        start(0, 0)

        def chunk(j, carry):
            b = lax.rem(j, 2)

            @pl.when(j < KCH - 1)
            def _():
                start(j + 1, 1 - b)

            finish(j, b)
            return carry

        lax.fori_loop(0, KCH, chunk, 0)
        plsc.subcore_barrier()

        for k in range(NSLAB):
            r0 = base + k * RSLAB
            pltpu.sync_copy(s_sh.at[pl.ds(r0, RSLAB), :], st_v)
            pltpu.sync_copy(st_v, s_out.at[c, pl.ds(r0, RSLAB), :])

    return edge_pass


_DOT = functools.partial(jnp.dot, preferred_element_type=jnp.float32)


def _mm_body(x_ref, w_ref, o_ref):
    o_ref[...] = _DOT(x_ref[...], w_ref[...])


def _make_mm(rows, din, dout):
    return pl.pallas_call(
        _mm_body, out_shape=jax.ShapeDtypeStruct((rows, dout), jnp.float32))


_BN = 2000


def _make_layer(din, dout, gout, act):
    def body(h_ref, ws_ref, sp_ref, ep_ref, dp_ref, we_ref, b_ref, *rest):
        if gout is not None:
            wm_ref, h_out, g_out = rest
        else:
            (h_out,) = rest
        S = sp_ref[0] + sp_ref[1]
        Eg = ep_ref[0] + ep_ref[1]
        deg = dp_ref[0, :, 0:1] + dp_ref[1, :, 0:1]
        deg = jnp.maximum(deg, 1.0)
        agg = (S + _DOT(Eg, we_ref[...])) / deg
        h = _DOT(h_ref[...], ws_ref[...]) + agg + b_ref[...]
        h = act(h)
        h_out[...] = h
        if gout is not None:
            g_out[...] = _DOT(h, wm_ref[...])

    row_blk = lambda w: pl.BlockSpec((_BN, w), lambda i: (i, 0))
    part_blk = lambda w: pl.BlockSpec((2, _BN, w), lambda i: (0, i, 0))
    full = lambda a, b: pl.BlockSpec((a, b), lambda i: (0, 0))
    in_specs = [row_blk(din), full(din, dout), part_blk(dout),
                part_blk(16), part_blk(16), full(16, dout), full(1, dout)]
    out_specs = [row_blk(dout)]
    out_shape = [jax.ShapeDtypeStruct((N, dout), jnp.float32)]
    if gout is not None:
        in_specs.append(full(dout, gout))
        out_specs.append(row_blk(gout))
        out_shape.append(jax.ShapeDtypeStruct((N, gout), jnp.float32))
    return pl.pallas_call(
        body, grid=(N // _BN,), in_specs=in_specs,
        out_specs=out_specs if gout is not None else out_specs[0],
        out_shape=out_shape if gout is not None else out_shape[0])


_edge_pass0 = functools.cache(_make_edge_pass0)
_edge_pass_h = functools.cache(lambda: _make_edge_pass(H))
_edge_pass_16 = functools.cache(lambda: _make_edge_pass(16))
_MM0 = _make_mm(N, DF, H)
_LAYER1 = _make_layer(DF, H, H, jax.nn.relu)
_LAYER2 = _make_layer(H, H, 16, jax.nn.relu)
_LAYER3 = _make_layer(H, 16, None, jnp.tanh)


def kernel(data, edge_index, efeat, Wm0, We0, Ws0, b0,
           Wm1, We1, Ws1, b1, Wm2, We2, Ws2, b2):
    Wm2p = jnp.pad(Wm2, ((0, 0), (0, 15)))
    We2p = jnp.pad(We2, ((0, 0), (0, 15)))
    Ws2p = jnp.pad(Ws2, ((0, 0), (0, 15)))
    b2p = jnp.pad(b2, (0, 15)).reshape(1, 16)
    b0r = b0.reshape(1, H)
    b1r = b1.reshape(1, H)

    g0 = _MM0(data, Wm0)
    Sp0, Ep, Dp = _edge_pass0()(edge_index, efeat, g0)
    h1, g1 = _LAYER1(data, Ws0, Sp0, Ep, Dp, We0, b0r, Wm1)
    Sp1 = _edge_pass_h()(edge_index, g1)
    h2, g2 = _LAYER2(h1, Ws1, Sp1, Ep, Dp, We1, b1r, Wm2p)
    Sp2 = _edge_pass_16()(edge_index, g2)
    h3 = _LAYER3(h2, Ws2p, Sp2, Ep, Dp, We2p, b2p)
    return h3[0:1, 0]


# speedup vs baseline: 1.2822x; 1.2822x over previous
import functools

import jax
import jax.numpy as jnp
from jax import lax
from jax.experimental import pallas as pl
from jax.experimental.pallas import tpu as pltpu
from jax.experimental.pallas import tpu_sc as plsc

N = 10000
E = 320000
DF = 128
H = 64
NCORES = 2
NSUB = 16
NW = NCORES * NSUB
EPT = E // NW
CH = 200
KCH = EPT // CH
NPAD = 10240
RPT = NPAD // NSUB
RSLAB = 128
NSLAB = RPT // RSLAB


def _sc_mesh():
    return plsc.VectorSubcoreMesh(
        core_axis_name="c", subcore_axis_name="s",
        num_cores=NCORES, num_subcores=NSUB)


_SC_PARAMS = pltpu.CompilerParams(use_tc_tiling_on_sc=False)


def _fill_rows(ref, nrows, ncols, value):
    vec = jnp.full((16,), value, jnp.float32)

    def body(i, carry):
        for k in range(ncols // 16):
            ref[i, pl.ds(k * 16, 16)] = vec
        return carry

    lax.fori_loop(0, nrows, body, 0)


def _make_edge_pass0():
    out_type = [
        jax.ShapeDtypeStruct((NCORES, NPAD, 128), jnp.float32),
        jax.ShapeDtypeStruct((NCORES, NPAD, 128), jnp.float32),
        jax.ShapeDtypeStruct((NCORES, NPAD, 128), jnp.float32),
    ]
    scratch = [
        pltpu.VMEM((EPT,), jnp.int32),
        pltpu.VMEM((EPT,), jnp.int32),
        pltpu.VMEM((2, CH, H), jnp.float32),
        pltpu.VMEM((2, CH, 16), jnp.float32),
        pltpu.VMEM((CH, 16), jnp.float32),
        pltpu.VMEM((RSLAB, H), jnp.float32),
        pltpu.VMEM((RSLAB, 16), jnp.float32),
        pltpu.VMEM_SHARED((NPAD, H), jnp.float32),
        pltpu.VMEM_SHARED((NPAD, 16), jnp.float32),
        pltpu.VMEM_SHARED((NPAD, 16), jnp.float32),
        pltpu.SemaphoreType.DMA((2,)),
        pltpu.SemaphoreType.DMA((2,)),
    ]

    @functools.partial(pl.kernel, out_type=out_type, mesh=_sc_mesh(),
                       scratch_types=scratch, compiler_params=_SC_PARAMS)
    def edge_pass0(ei_hbm, ef_hbm, g_hbm,
                   s_out, e_out, d_out,
                   src_v, dst_v, rows_v, ef_v, ones_v, st_w, st_n,
                   s_sh, e_sh, d_sh, gsem, esem):
        c = lax.axis_index("c")
        s = lax.axis_index("s")
        wid = s * NCORES + c
        ebase = pl.multiple_of(wid * EPT, CH)
        base = pl.multiple_of(s * RPT, RSLAB)
        _fill_rows(st_w, RSLAB, H, 0.0)
        _fill_rows(st_n, RSLAB, 16, 0.0)
        for k in range(NSLAB):
            r0 = base + k * RSLAB
            pltpu.sync_copy(st_w, s_sh.at[pl.ds(r0, RSLAB), :])
            pltpu.sync_copy(st_n, e_sh.at[pl.ds(r0, RSLAB), :])
            pltpu.sync_copy(st_n, d_sh.at[pl.ds(r0, RSLAB), :])
        _fill_rows(ones_v, CH, 16, 1.0)
        plsc.subcore_barrier()

        pltpu.sync_copy(ei_hbm.at[0, pl.ds(ebase, EPT)], src_v)
        pltpu.sync_copy(ei_hbm.at[1, pl.ds(ebase, EPT)], dst_v)

        def start(j, b):
            o = pl.multiple_of(j * CH, CH)
            pltpu.async_copy(g_hbm.at[src_v.at[pl.ds(o, CH)]], rows_v.at[b],
                             gsem.at[b])
            pltpu.async_copy(ef_hbm.at[pl.ds(ebase + o, CH), :], ef_v.at[b],
                             esem.at[b])

        def finish(j, b):
            o = pl.multiple_of(j * CH, CH)
            pltpu.make_async_copy(g_hbm.at[src_v.at[pl.ds(o, CH)]],
                                  rows_v.at[b], gsem.at[b]).wait()
            pltpu.make_async_copy(ef_hbm.at[pl.ds(ebase + o, CH), :],
                                  ef_v.at[b], esem.at[b]).wait()
            dsts = dst_v.at[pl.ds(o, CH)]
            pltpu.sync_copy(rows_v.at[b], s_sh.at[dsts], add=True)
            pltpu.sync_copy(ef_v.at[b], e_sh.at[dsts], add=True)
            pltpu.sync_copy(ones_v, d_sh.at[dsts], add=True)

        start(0, 0)

        def chunk(j, carry):
            b = lax.rem(j, 2)

            @pl.when(j < KCH - 1)
            def _():
                start(j + 1, 1 - b)

            finish(j, b)
            return carry

        lax.fori_loop(0, KCH, chunk, 0)
        plsc.subcore_barrier()

        for k in range(NSLAB):
            r0 = base + k * RSLAB
            pltpu.sync_copy(s_sh.at[pl.ds(r0, RSLAB), :], st_w)
            pltpu.sync_copy(st_w, s_out.at[c, pl.ds(r0, RSLAB), 0:H])
            pltpu.sync_copy(e_sh.at[pl.ds(r0, RSLAB), :], st_n)
            pltpu.sync_copy(st_n, e_out.at[c, pl.ds(r0, RSLAB), 0:16])
            pltpu.sync_copy(d_sh.at[pl.ds(r0, RSLAB), :], st_n)
            pltpu.sync_copy(st_n, d_out.at[c, pl.ds(r0, RSLAB), 0:16])

    return edge_pass0


def _make_edge_pass(width):
    out_type = jax.ShapeDtypeStruct((NCORES, NPAD, 128), jnp.float32)
    scratch = [
        pltpu.VMEM((EPT,), jnp.int32),
        pltpu.VMEM((EPT,), jnp.int32),
        pltpu.VMEM((2, CH, width), jnp.float32),
        pltpu.VMEM((RSLAB, width), jnp.float32),
        pltpu.VMEM_SHARED((NPAD, width), jnp.float32),
        pltpu.SemaphoreType.DMA((2,)),
    ]

    @functools.partial(pl.kernel, out_type=out_type, mesh=_sc_mesh(),
                       scratch_types=scratch, compiler_params=_SC_PARAMS)
    def edge_pass(ei_hbm, g_hbm, s_out,
                  src_v, dst_v, rows_v, st_v, s_sh, gsem):
        c = lax.axis_index("c")
        s = lax.axis_index("s")
        wid = s * NCORES + c
        ebase = pl.multiple_of(wid * EPT, CH)
        base = pl.multiple_of(s * RPT, RSLAB)
        _fill_rows(st_v, RSLAB, width, 0.0)
        for k in range(NSLAB):
            pltpu.sync_copy(st_v, s_sh.at[pl.ds(base + k * RSLAB, RSLAB), :])
        plsc.subcore_barrier()

        pltpu.sync_copy(ei_hbm.at[0, pl.ds(ebase, EPT)], src_v)
        pltpu.sync_copy(ei_hbm.at[1, pl.ds(ebase, EPT)], dst_v)

        def start(j, b):
            o = pl.multiple_of(j * CH, CH)
            pltpu.async_copy(g_hbm.at[src_v.at[pl.ds(o, CH)]], rows_v.at[b],
                             gsem.at[b])

        def finish(j, b):
            o = pl.multiple_of(j * CH, CH)
            pltpu.make_async_copy(g_hbm.at[src_v.at[pl.ds(o, CH)]],
                                  rows_v.at[b], gsem.at[b]).wait()
            pltpu.sync_copy(rows_v.at[b], s_sh.at[dst_v.at[pl.ds(o, CH)]],
                            add=True)

        start(0, 0)

        def chunk(j, carry):
            b = lax.rem(j, 2)

            @pl.when(j < KCH - 1)
            def _():
                start(j + 1, 1 - b)

            finish(j, b)
            return carry

        lax.fori_loop(0, KCH, chunk, 0)
        plsc.subcore_barrier()

        for k in range(NSLAB):
            r0 = base + k * RSLAB
            pltpu.sync_copy(s_sh.at[pl.ds(r0, RSLAB), :], st_v)
            pltpu.sync_copy(st_v, s_out.at[c, pl.ds(r0, RSLAB), 0:width])

    return edge_pass


_DOT = functools.partial(jnp.dot, preferred_element_type=jnp.float32)


def _mm_body(x_ref, w_ref, o_ref):
    o_ref[...] = _DOT(x_ref[...], w_ref[...])


def _make_mm(rows, din, dout):
    return pl.pallas_call(
        _mm_body, out_shape=jax.ShapeDtypeStruct((rows, dout), jnp.float32))


_BN = 2000


def _make_layer(din, dout, gout, act):
    def body(h_ref, ws_ref, sp_ref, ep_ref, dp_ref, we_ref, b_ref, *rest):
        if gout is not None:
            wm_ref, h_out, g_out = rest
        else:
            (h_out,) = rest
        S = sp_ref[0, :, 0:dout] + sp_ref[1, :, 0:dout]
        Eg = ep_ref[0, :, 0:16] + ep_ref[1, :, 0:16]
        deg = dp_ref[0, :, 0:1] + dp_ref[1, :, 0:1]
        deg = jnp.maximum(deg, 1.0)
        agg = (S + _DOT(Eg, we_ref[...])) / deg
        h = _DOT(h_ref[...], ws_ref[...]) + agg + b_ref[...]
        h = act(h)
        h_out[...] = h
        if gout is not None:
            g_out[...] = _DOT(h, wm_ref[...])

    row_blk = lambda w: pl.BlockSpec((_BN, w), lambda i: (i, 0))
    part_blk = lambda w: pl.BlockSpec((2, _BN, w), lambda i: (0, i, 0))
    full = lambda a, b: pl.BlockSpec((a, b), lambda i: (0, 0))
    in_specs = [row_blk(din), full(din, dout), part_blk(128),
                part_blk(128), part_blk(128), full(16, dout), full(1, dout)]
    out_specs = [row_blk(dout)]
    out_shape = [jax.ShapeDtypeStruct((N, dout), jnp.float32)]
    if gout is not None:
        in_specs.append(full(dout, gout))
        out_specs.append(row_blk(gout))
        out_shape.append(jax.ShapeDtypeStruct((N, gout), jnp.float32))
    return pl.pallas_call(
        body, grid=(N // _BN,), in_specs=in_specs,
        out_specs=out_specs if gout is not None else out_specs[0],
        out_shape=out_shape if gout is not None else out_shape[0])


_edge_pass0 = functools.cache(_make_edge_pass0)
_edge_pass_h = functools.cache(lambda: _make_edge_pass(H))
_edge_pass_16 = functools.cache(lambda: _make_edge_pass(16))
_MM0 = _make_mm(N, DF, H)
_LAYER1 = _make_layer(DF, H, H, jax.nn.relu)
_LAYER2 = _make_layer(H, H, 16, jax.nn.relu)
_LAYER3 = _make_layer(H, 16, None, jnp.tanh)


def kernel(data, edge_index, efeat, Wm0, We0, Ws0, b0,
           Wm1, We1, Ws1, b1, Wm2, We2, Ws2, b2):
    Wm2p = jnp.pad(Wm2, ((0, 0), (0, 15)))
    We2p = jnp.pad(We2, ((0, 0), (0, 15)))
    Ws2p = jnp.pad(Ws2, ((0, 0), (0, 15)))
    b2p = jnp.pad(b2, (0, 15)).reshape(1, 16)
    b0r = b0.reshape(1, H)
    b1r = b1.reshape(1, H)

    g0 = _MM0(data, Wm0)
    Sp0, Ep, Dp = _edge_pass0()(edge_index, efeat, g0)
    h1, g1 = _LAYER1(data, Ws0, Sp0, Ep, Dp, We0, b0r, Wm1)
    Sp1 = _edge_pass_h()(edge_index, g1)
    h2, g2 = _LAYER2(h1, Ws1, Sp1, Ep, Dp, We1, b1r, Wm2p)
    Sp2 = _edge_pass_16()(edge_index, g2)
    h3 = _LAYER3(h2, Ws2p, Sp2, Ep, Dp, We2p, b2p)
    return h3[0:1, 0]


# speedup vs baseline: 1.4597x; 1.1384x over previous
import functools

import jax
import jax.numpy as jnp
from jax import lax
from jax.experimental import pallas as pl
from jax.experimental.pallas import tpu as pltpu
from jax.experimental.pallas import tpu_sc as plsc

N = 10000
E = 320000
DF = 128
H = 64
NCORES = 2
NSUB = 16
NW = NCORES * NSUB
EPT = E // NW
CH = 200
KCH = EPT // CH
NPAD = 10240
RPT = NPAD // NSUB
RSLAB = 128
NSLAB = RPT // RSLAB


def _sc_mesh():
    return plsc.VectorSubcoreMesh(
        core_axis_name="c", subcore_axis_name="s",
        num_cores=NCORES, num_subcores=NSUB)


_SC_PARAMS = pltpu.CompilerParams(use_tc_tiling_on_sc=False)


def _fill_rows(ref, nrows, ncols, value):
    vec = jnp.full((16,), value, jnp.float32)

    def body(i, carry):
        for k in range(ncols // 16):
            ref[i, pl.ds(k * 16, 16)] = vec
        return carry

    lax.fori_loop(0, nrows, body, 0)


def _make_edge_pass_ef():
    out_type = jax.ShapeDtypeStruct((NCORES, NPAD, 128), jnp.float32)
    scratch = [
        pltpu.VMEM((EPT,), jnp.int32),
        pltpu.VMEM((2, CH, 16), jnp.float32),
        pltpu.VMEM((CH, 16), jnp.float32),
        pltpu.VMEM((RSLAB, 16), jnp.float32),
        pltpu.VMEM_SHARED((NPAD, 16), jnp.float32),
        pltpu.VMEM_SHARED((NPAD, 16), jnp.float32),
        pltpu.SemaphoreType.DMA((2,)),
    ]

    @functools.partial(pl.kernel, out_type=out_type, mesh=_sc_mesh(),
                       scratch_types=scratch, compiler_params=_SC_PARAMS)
    def edge_pass_ef(ei_hbm, ef_hbm, e_out,
                     dst_v, ef_v, ones_v, st_n, e_sh, d_sh, esem):
        c = lax.axis_index("c")
        s = lax.axis_index("s")
        wid = s * NCORES + c
        ebase = pl.multiple_of(wid * EPT, CH)
        base = pl.multiple_of(s * RPT, RSLAB)
        _fill_rows(st_n, RSLAB, 16, 0.0)
        for k in range(NSLAB):
            r0 = base + k * RSLAB
            pltpu.sync_copy(st_n, e_sh.at[pl.ds(r0, RSLAB), :])
            pltpu.sync_copy(st_n, d_sh.at[pl.ds(r0, RSLAB), :])
        _fill_rows(ones_v, CH, 16, 1.0)
        plsc.subcore_barrier()

        pltpu.sync_copy(ei_hbm.at[1, pl.ds(ebase, EPT)], dst_v)

        def start(j, b):
            o = pl.multiple_of(j * CH, CH)
            pltpu.async_copy(ef_hbm.at[pl.ds(ebase + o, CH), :], ef_v.at[b],
                             esem.at[b])

        def finish(j, b):
            o = pl.multiple_of(j * CH, CH)
            pltpu.make_async_copy(ef_hbm.at[pl.ds(ebase + o, CH), :],
                                  ef_v.at[b], esem.at[b]).wait()
            dsts = dst_v.at[pl.ds(o, CH)]
            pltpu.sync_copy(ef_v.at[b], e_sh.at[dsts], add=True)
            pltpu.sync_copy(ones_v, d_sh.at[dsts], add=True)

        start(0, 0)

        def chunk(j, carry):
            b = lax.rem(j, 2)

            @pl.when(j < KCH - 1)
            def _():
                start(j + 1, 1 - b)

            finish(j, b)
            return carry

        lax.fori_loop(0, KCH, chunk, 0)
        plsc.subcore_barrier()

        for k in range(NSLAB):
            r0 = base + k * RSLAB
            pltpu.sync_copy(e_sh.at[pl.ds(r0, RSLAB), :], st_n)
            pltpu.sync_copy(st_n, e_out.at[c, pl.ds(r0, RSLAB), 0:16])
            pltpu.sync_copy(d_sh.at[pl.ds(r0, RSLAB), :], st_n)
            pltpu.sync_copy(st_n, e_out.at[c, pl.ds(r0, RSLAB), 16:32])

    return edge_pass_ef


def _make_edge_pass(width):
    out_type = jax.ShapeDtypeStruct((NCORES, NPAD, 128), jnp.float32)
    scratch = [
        pltpu.VMEM((EPT,), jnp.int32),
        pltpu.VMEM((EPT,), jnp.int32),
        pltpu.VMEM((2, CH, width), jnp.float32),
        pltpu.VMEM((RSLAB, width), jnp.float32),
        pltpu.VMEM_SHARED((NPAD, width), jnp.float32),
        pltpu.SemaphoreType.DMA((2,)),
    ]

    @functools.partial(pl.kernel, out_type=out_type, mesh=_sc_mesh(),
                       scratch_types=scratch, compiler_params=_SC_PARAMS)
    def edge_pass(ei_hbm, g_hbm, s_out,
                  src_v, dst_v, rows_v, st_v, s_sh, gsem):
        c = lax.axis_index("c")
        s = lax.axis_index("s")
        wid = s * NCORES + c
        ebase = pl.multiple_of(wid * EPT, CH)
        base = pl.multiple_of(s * RPT, RSLAB)
        _fill_rows(st_v, RSLAB, width, 0.0)
        for k in range(NSLAB):
            pltpu.sync_copy(st_v, s_sh.at[pl.ds(base + k * RSLAB, RSLAB), :])
        plsc.subcore_barrier()

        pltpu.sync_copy(ei_hbm.at[0, pl.ds(ebase, EPT)], src_v)
        pltpu.sync_copy(ei_hbm.at[1, pl.ds(ebase, EPT)], dst_v)

        def start(j, b):
            o = pl.multiple_of(j * CH, CH)
            pltpu.async_copy(g_hbm.at[src_v.at[pl.ds(o, CH)]], rows_v.at[b],
                             gsem.at[b])

        def finish(j, b):
            o = pl.multiple_of(j * CH, CH)
            pltpu.make_async_copy(g_hbm.at[src_v.at[pl.ds(o, CH)]],
                                  rows_v.at[b], gsem.at[b]).wait()
            pltpu.sync_copy(rows_v.at[b], s_sh.at[dst_v.at[pl.ds(o, CH)]],
                            add=True)

        start(0, 0)

        def chunk(j, carry):
            b = lax.rem(j, 2)

            @pl.when(j < KCH - 1)
            def _():
                start(j + 1, 1 - b)

            finish(j, b)
            return carry

        lax.fori_loop(0, KCH, chunk, 0)
        plsc.subcore_barrier()

        for k in range(NSLAB):
            r0 = base + k * RSLAB
            pltpu.sync_copy(s_sh.at[pl.ds(r0, RSLAB), :], st_v)
            pltpu.sync_copy(st_v, s_out.at[c, pl.ds(r0, RSLAB), 0:width])

    return edge_pass


_DOT = functools.partial(jnp.dot, preferred_element_type=jnp.float32)


def _mm_body(x_ref, w_ref, o_ref):
    o_ref[...] = _DOT(x_ref[...], w_ref[...])


def _make_mm(rows, din, dout):
    return pl.pallas_call(
        _mm_body, out_shape=jax.ShapeDtypeStruct((rows, dout), jnp.float32))


_BN = 2000


def _make_layer(din, dout, gout, act, rows=N, bn=_BN):
    def body(h_ref, ws_ref, sp_ref, ep_ref, we_ref, b_ref, *rest):
        if gout is not None:
            wm_ref, h_out, g_out = rest
        else:
            (h_out,) = rest
        S = sp_ref[0, :, 0:dout] + sp_ref[1, :, 0:dout]
        Eg = ep_ref[0, :, 0:16] + ep_ref[1, :, 0:16]
        deg = ep_ref[0, :, 16:17] + ep_ref[1, :, 16:17]
        deg = jnp.maximum(deg, 1.0)
        agg = (S + _DOT(Eg, we_ref[...])) / deg
        h = _DOT(h_ref[...], ws_ref[...]) + agg + b_ref[...]
        h = act(h)
        h_out[...] = h
        if gout is not None:
            g_out[...] = _DOT(h, wm_ref[...])

    row_blk = lambda w: pl.BlockSpec((bn, w), lambda i: (i, 0))
    part_blk = pl.BlockSpec((2, bn, 128), lambda i: (0, i, 0))
    full = lambda a, b: pl.BlockSpec((a, b), lambda i: (0, 0))
    in_specs = [row_blk(din), full(din, dout), part_blk,
                part_blk, full(16, dout), full(1, dout)]
    out_specs = [row_blk(dout)]
    out_shape = [jax.ShapeDtypeStruct((rows, dout), jnp.float32)]
    if gout is not None:
        in_specs.append(full(dout, gout))
        out_specs.append(row_blk(gout))
        out_shape.append(jax.ShapeDtypeStruct((rows, gout), jnp.float32))
    return pl.pallas_call(
        body, grid=(rows // bn,), in_specs=in_specs,
        out_specs=out_specs if gout is not None else out_specs[0],
        out_shape=out_shape if gout is not None else out_shape[0])


_edge_pass_ef = functools.cache(_make_edge_pass_ef)
_edge_pass_h = functools.cache(lambda: _make_edge_pass(H))
_edge_pass_16 = functools.cache(lambda: _make_edge_pass(16))
_MM0 = _make_mm(N, DF, H)
_LAYER1 = _make_layer(DF, H, H, jax.nn.relu)
_LAYER2 = _make_layer(H, H, 16, jax.nn.relu)
_LAYER3 = _make_layer(H, 16, None, jnp.tanh, rows=8, bn=8)


def kernel(data, edge_index, efeat, Wm0, We0, Ws0, b0,
           Wm1, We1, Ws1, b1, Wm2, We2, Ws2, b2):
    Wm2p = jnp.pad(Wm2, ((0, 0), (0, 15)))
    We2p = jnp.pad(We2, ((0, 0), (0, 15)))
    Ws2p = jnp.pad(Ws2, ((0, 0), (0, 15)))
    b2p = jnp.pad(b2, (0, 15)).reshape(1, 16)
    b0r = b0.reshape(1, H)
    b1r = b1.reshape(1, H)

    g0 = _MM0(data, Wm0)
    Sp0 = _edge_pass_h()(edge_index, g0)
    Ep = _edge_pass_ef()(edge_index, efeat)
    h1, g1 = _LAYER1(data, Ws0, Sp0, Ep, We0, b0r, Wm1)
    Sp1 = _edge_pass_h()(edge_index, g1)
    h2, g2 = _LAYER2(h1, Ws1, Sp1, Ep, We1, b1r, Wm2p)
    Sp2 = _edge_pass_16()(edge_index, g2)
    h3 = _LAYER3(h2[0:8], Ws2p, Sp2, Ep, We2p, b2p)
    return h3[0:1, 0]


# speedup vs baseline: 1.5180x; 1.0399x over previous
import functools

import jax
import jax.numpy as jnp
from jax import lax
from jax.experimental import pallas as pl
from jax.experimental.pallas import tpu as pltpu
from jax.experimental.pallas import tpu_sc as plsc

N = 10000
E = 320000
DF = 128
H = 64
NCORES = 2
NSUB = 16
NW = NCORES * NSUB
EPT = E // NW
CH_W = 400
CH_N = 1000
NPAD = 10240
RPT = NPAD // NSUB
RSLAB = 128
NSLAB = RPT // RSLAB


def _sc_mesh():
    return plsc.VectorSubcoreMesh(
        core_axis_name="c", subcore_axis_name="s",
        num_cores=NCORES, num_subcores=NSUB)


_SC_PARAMS = pltpu.CompilerParams(use_tc_tiling_on_sc=False)


def _fill_rows(ref, nrows, ncols, value):
    vec = jnp.full((16,), value, jnp.float32)

    def body(i, carry):
        for k in range(ncols // 16):
            ref[i, pl.ds(k * 16, 16)] = vec
        return carry

    lax.fori_loop(0, nrows, body, 0)


def _make_edge_pass_ef():
    CH = CH_N
    KCH = EPT // CH
    out_type = jax.ShapeDtypeStruct((NCORES, NPAD, 128), jnp.float32)
    scratch = [
        pltpu.VMEM((EPT,), jnp.int32),
        pltpu.VMEM((2, CH, 16), jnp.float32),
        pltpu.VMEM((CH, 16), jnp.float32),
        pltpu.VMEM((RSLAB, 16), jnp.float32),
        pltpu.VMEM_SHARED((NPAD, 16), jnp.float32),
        pltpu.VMEM_SHARED((NPAD, 16), jnp.float32),
        pltpu.SemaphoreType.DMA((2,)),
    ]

    @functools.partial(pl.kernel, out_type=out_type, mesh=_sc_mesh(),
                       scratch_types=scratch, compiler_params=_SC_PARAMS)
    def edge_pass_ef(ei_hbm, ef_hbm, e_out,
                     dst_v, ef_v, ones_v, st_n, e_sh, d_sh, esem):
        c = lax.axis_index("c")
        s = lax.axis_index("s")
        wid = s * NCORES + c
        ebase = pl.multiple_of(wid * EPT, CH)
        base = pl.multiple_of(s * RPT, RSLAB)
        _fill_rows(st_n, RSLAB, 16, 0.0)
        for k in range(NSLAB):
            r0 = base + k * RSLAB
            pltpu.sync_copy(st_n, e_sh.at[pl.ds(r0, RSLAB), :])
            pltpu.sync_copy(st_n, d_sh.at[pl.ds(r0, RSLAB), :])
        _fill_rows(ones_v, CH, 16, 1.0)
        plsc.subcore_barrier()

        pltpu.sync_copy(ei_hbm.at[1, pl.ds(ebase, EPT)], dst_v)

        def start(j, b):
            o = pl.multiple_of(j * CH, CH)
            pltpu.async_copy(ef_hbm.at[pl.ds(ebase + o, CH), :], ef_v.at[b],
                             esem.at[b])

        def finish(j, b):
            o = pl.multiple_of(j * CH, CH)
            pltpu.make_async_copy(ef_hbm.at[pl.ds(ebase + o, CH), :],
                                  ef_v.at[b], esem.at[b]).wait()
            dsts = dst_v.at[pl.ds(o, CH)]
            pltpu.sync_copy(ef_v.at[b], e_sh.at[dsts], add=True)
            pltpu.sync_copy(ones_v, d_sh.at[dsts], add=True)

        start(0, 0)

        def chunk(j, carry):
            b = lax.rem(j, 2)

            @pl.when(j < KCH - 1)
            def _():
                start(j + 1, 1 - b)

            finish(j, b)
            return carry

        lax.fori_loop(0, KCH, chunk, 0)
        plsc.subcore_barrier()

        for k in range(NSLAB):
            r0 = base + k * RSLAB
            pltpu.sync_copy(e_sh.at[pl.ds(r0, RSLAB), :], st_n)
            pltpu.sync_copy(st_n, e_out.at[c, pl.ds(r0, RSLAB), 0:16])
            pltpu.sync_copy(d_sh.at[pl.ds(r0, RSLAB), :], st_n)
            pltpu.sync_copy(st_n, e_out.at[c, pl.ds(r0, RSLAB), 16:32])

    return edge_pass_ef


def _make_edge_pass(width):
    CH = CH_W if width > 16 else CH_N
    KCH = EPT // CH
    out_type = jax.ShapeDtypeStruct((NCORES, NPAD, 128), jnp.float32)
    scratch = [
        pltpu.VMEM((EPT,), jnp.int32),
        pltpu.VMEM((EPT,), jnp.int32),
        pltpu.VMEM((2, CH, width), jnp.float32),
        pltpu.VMEM((RSLAB, width), jnp.float32),
        pltpu.VMEM_SHARED((NPAD, width), jnp.float32),
        pltpu.SemaphoreType.DMA((2,)),
    ]

    @functools.partial(pl.kernel, out_type=out_type, mesh=_sc_mesh(),
                       scratch_types=scratch, compiler_params=_SC_PARAMS)
    def edge_pass(ei_hbm, g_hbm, s_out,
                  src_v, dst_v, rows_v, st_v, s_sh, gsem):
        c = lax.axis_index("c")
        s = lax.axis_index("s")
        wid = s * NCORES + c
        ebase = pl.multiple_of(wid * EPT, CH)
        base = pl.multiple_of(s * RPT, RSLAB)
        _fill_rows(st_v, RSLAB, width, 0.0)
        for k in range(NSLAB):
            pltpu.sync_copy(st_v, s_sh.at[pl.ds(base + k * RSLAB, RSLAB), :])
        plsc.subcore_barrier()

        pltpu.sync_copy(ei_hbm.at[0, pl.ds(ebase, EPT)], src_v)
        pltpu.sync_copy(ei_hbm.at[1, pl.ds(ebase, EPT)], dst_v)

        def start(j, b):
            o = pl.multiple_of(j * CH, CH)
            pltpu.async_copy(g_hbm.at[src_v.at[pl.ds(o, CH)]], rows_v.at[b],
                             gsem.at[b])

        def finish(j, b):
            o = pl.multiple_of(j * CH, CH)
            pltpu.make_async_copy(g_hbm.at[src_v.at[pl.ds(o, CH)]],
                                  rows_v.at[b], gsem.at[b]).wait()
            pltpu.sync_copy(rows_v.at[b], s_sh.at[dst_v.at[pl.ds(o, CH)]],
                            add=True)

        start(0, 0)

        def chunk(j, carry):
            b = lax.rem(j, 2)

            @pl.when(j < KCH - 1)
            def _():
                start(j + 1, 1 - b)

            finish(j, b)
            return carry

        lax.fori_loop(0, KCH, chunk, 0)
        plsc.subcore_barrier()

        for k in range(NSLAB):
            r0 = base + k * RSLAB
            pltpu.sync_copy(s_sh.at[pl.ds(r0, RSLAB), :], st_v)
            pltpu.sync_copy(st_v, s_out.at[c, pl.ds(r0, RSLAB), 0:width])

    return edge_pass


_DOT = functools.partial(jnp.dot, preferred_element_type=jnp.float32)


def _mm_body(x_ref, w_ref, o_ref):
    o_ref[...] = _DOT(x_ref[...], w_ref[...])


def _make_mm(rows, din, dout):
    return pl.pallas_call(
        _mm_body, out_shape=jax.ShapeDtypeStruct((rows, dout), jnp.float32))


_BN = 2000


def _make_layer(din, dout, gout, act, rows=N, bn=_BN):
    def body(h_ref, ws_ref, sp_ref, ep_ref, we_ref, b_ref, *rest):
        if gout is not None:
            wm_ref, h_out, g_out = rest
        else:
            (h_out,) = rest
        S = sp_ref[0, :, 0:dout] + sp_ref[1, :, 0:dout]
        Eg = ep_ref[0, :, 0:16] + ep_ref[1, :, 0:16]
        deg = ep_ref[0, :, 16:17] + ep_ref[1, :, 16:17]
        deg = jnp.maximum(deg, 1.0)
        agg = (S + jnp.dot(Eg, we_ref[...], precision=lax.Precision.HIGHEST,
                           preferred_element_type=jnp.float32)) / deg
        h = _DOT(h_ref[...], ws_ref[...]) + agg + b_ref[...]
        h = act(h)
        h_out[...] = h
        if gout is not None:
            g_out[...] = _DOT(h, wm_ref[...])

    row_blk = lambda w: pl.BlockSpec((bn, w), lambda i: (i, 0))
    part_blk = pl.BlockSpec((2, bn, 128), lambda i: (0, i, 0))
    full = lambda a, b: pl.BlockSpec((a, b), lambda i: (0, 0))
    in_specs = [row_blk(din), full(din, dout), part_blk,
                part_blk, full(16, dout), full(1, dout)]
    out_specs = [row_blk(dout)]
    out_shape = [jax.ShapeDtypeStruct((rows, dout), jnp.float32)]
    if gout is not None:
        in_specs.append(full(dout, gout))
        out_specs.append(row_blk(gout))
        out_shape.append(jax.ShapeDtypeStruct((rows, gout), jnp.float32))
    return pl.pallas_call(
        body, grid=(rows // bn,), in_specs=in_specs,
        out_specs=out_specs if gout is not None else out_specs[0],
        out_shape=out_shape if gout is not None else out_shape[0])


_edge_pass_ef = functools.cache(_make_edge_pass_ef)
_edge_pass_h = functools.cache(lambda: _make_edge_pass(H))
_edge_pass_16 = functools.cache(lambda: _make_edge_pass(16))
_MM0 = _make_mm(N, DF, H)
_LAYER1 = _make_layer(DF, H, H, jax.nn.relu)
_LAYER2 = _make_layer(H, H, 16, jax.nn.relu)
_LAYER3 = _make_layer(H, 16, None, jnp.tanh, rows=8, bn=8)


def kernel(data, edge_index, efeat, Wm0, We0, Ws0, b0,
           Wm1, We1, Ws1, b1, Wm2, We2, Ws2, b2):
    Wm2p = jnp.pad(Wm2, ((0, 0), (0, 15)))
    We2p = jnp.pad(We2, ((0, 0), (0, 15)))
    Ws2p = jnp.pad(Ws2, ((0, 0), (0, 15)))
    b2p = jnp.pad(b2, (0, 15)).reshape(1, 16)
    b0r = b0.reshape(1, H)
    b1r = b1.reshape(1, H)

    g0 = _MM0(data, Wm0)
    Sp0 = _edge_pass_h()(edge_index, g0)
    Ep = _edge_pass_ef()(edge_index, efeat)
    h1, g1 = _LAYER1(data, Ws0, Sp0, Ep, We0, b0r, Wm1)
    Sp1 = _edge_pass_h()(edge_index, g1)
    h2, g2 = _LAYER2(h1, Ws1, Sp1, Ep, We1, b1r, Wm2p)
    Sp2 = _edge_pass_16()(edge_index, g2)
    h3 = _LAYER3(h2[0:8], Ws2p, Sp2, Ep, We2p, b2p)
    return h3[0:1, 0]
